# baseline (speedup 1.0000x reference)
import jax
import jax.numpy as jnp
from jax.experimental import pallas as pl
from jax.experimental.pallas import tpu as pltpu

B, N, DIN, DOUT = 4, 2048, 128, 128
BN = 512
NG = N // BN
TOTAL = B * NG
NCH = 8
CH = N // NCH
NSLOT = 3
AHEAD = NSLOT - 1


def _gcn_body(x_ref, w_ref, adj_hbm, bias_ref, out_ref, sup_ref, abuf, sems):
    b = pl.program_id(0)
    g = pl.program_id(1)
    step = b * NG + g

    @pl.when(g == 0)
    def _():
        sup_ref[...] = jnp.dot(
            x_ref[0], w_ref[...], preferred_element_type=jnp.float32
        ).astype(jnp.bfloat16)

    def copy(k, i):
        kb = k // NG
        kg = k % NG
        return pltpu.make_async_copy(
            adj_hbm.at[kb, pl.ds(i * CH, CH), pl.ds(kg * BN, BN)],
            abuf.at[k % NSLOT, pl.ds(i * CH, CH), :],
            sems.at[k % NSLOT, i],
        )

    @pl.when(step == 0)
    def _():
        for k in range(AHEAD):
            for i in range(NCH):
                copy(k, i).start()

    @pl.when(step + AHEAD < TOTAL)
    def _():
        for i in range(NCH):
            copy(step + AHEAD, i).start()

    for i in range(NCH):
        copy(step, i).wait()

    out_ref[0] = jax.lax.dot_general(
        abuf[step % NSLOT].astype(jnp.bfloat16),
        sup_ref[...],
        (((0,), (0,)), ((), ())),
        preferred_element_type=jnp.float32,
    ) + bias_ref[...]


@jax.jit
def kernel(input, adj, weight, bias):
    bias2d = bias.reshape(1, DOUT)
    grid = (B, NG)
    return pl.pallas_call(
        _gcn_body,
        grid=grid,
        in_specs=[
            pl.BlockSpec((1, N, DIN), lambda b, g: (b, 0, 0)),
            pl.BlockSpec((DIN, DOUT), lambda b, g: (0, 0)),
            pl.BlockSpec(memory_space=pl.ANY),
            pl.BlockSpec((1, DOUT), lambda b, g: (0, 0)),
        ],
        out_specs=pl.BlockSpec((1, BN, DOUT), lambda b, g: (b, g, 0)),
        out_shape=jax.ShapeDtypeStruct((B, N, DOUT), jnp.float32),
        scratch_shapes=[
            pltpu.VMEM((N, DOUT), jnp.bfloat16),
            pltpu.VMEM((NSLOT, N, BN), jnp.float32),
            pltpu.SemaphoreType.DMA((NSLOT, NCH)),
        ],
        compiler_params=pltpu.CompilerParams(
            dimension_semantics=("arbitrary", "arbitrary"),
        ),
    )(input, weight, adj, bias2d)


# speedup vs baseline: 1.0326x; 1.0326x over previous
import jax
import jax.numpy as jnp
from jax.experimental import pallas as pl
from jax.experimental.pallas import tpu as pltpu

B, N, DIN, DOUT = 4, 2048, 128, 128
BN = 1024
NG = N // BN
TOTAL = B * NG
NCH = 8
CH = N // NCH
NSLOT = 2
AHEAD = NSLOT - 1


def _gcn_body(x_ref, w_ref, adj_hbm, bias_ref, out_ref, sup_ref, abuf, sems):
    b = pl.program_id(0)
    g = pl.program_id(1)
    step = b * NG + g

    @pl.when(g == 0)
    def _():
        sup_ref[...] = jnp.dot(
            x_ref[0], w_ref[...], preferred_element_type=jnp.float32
        ).astype(jnp.bfloat16)

    def copy(k, i):
        kb = k // NG
        kg = k % NG
        return pltpu.make_async_copy(
            adj_hbm.at[kb, pl.ds(i * CH, CH), pl.ds(kg * BN, BN)],
            abuf.at[k % NSLOT, pl.ds(i * CH, CH), :],
            sems.at[k % NSLOT, i],
        )

    @pl.when(step == 0)
    def _():
        for k in range(AHEAD):
            for i in range(NCH):
                copy(k, i).start()

    @pl.when(step + AHEAD < TOTAL)
    def _():
        for i in range(NCH):
            copy(step + AHEAD, i).start()

    for i in range(NCH):
        copy(step, i).wait()

    out_ref[0] = jax.lax.dot_general(
        abuf[step % NSLOT].astype(jnp.bfloat16),
        sup_ref[...],
        (((0,), (0,)), ((), ())),
        preferred_element_type=jnp.float32,
    ) + bias_ref[...]


@jax.jit
def kernel(input, adj, weight, bias):
    bias2d = bias.reshape(1, DOUT)
    grid = (B, NG)
    return pl.pallas_call(
        _gcn_body,
        grid=grid,
        in_specs=[
            pl.BlockSpec((1, N, DIN), lambda b, g: (b, 0, 0)),
            pl.BlockSpec((DIN, DOUT), lambda b, g: (0, 0)),
            pl.BlockSpec(memory_space=pl.ANY),
            pl.BlockSpec((1, DOUT), lambda b, g: (0, 0)),
        ],
        out_specs=pl.BlockSpec((1, BN, DOUT), lambda b, g: (b, g, 0)),
        out_shape=jax.ShapeDtypeStruct((B, N, DOUT), jnp.float32),
        scratch_shapes=[
            pltpu.VMEM((N, DOUT), jnp.bfloat16),
            pltpu.VMEM((NSLOT, N, BN), jnp.float32),
            pltpu.SemaphoreType.DMA((NSLOT, NCH)),
        ],
        compiler_params=pltpu.CompilerParams(
            dimension_semantics=("arbitrary", "arbitrary"),
        ),
    )(input, weight, adj, bias2d)
